# Initial kernel scaffold; baseline (speedup 1.0000x reference)
import jax
import jax.numpy as jnp
from jax.experimental import pallas as pl


def kernel(inp, clusters):
    raise NotImplementedError("write your pallas kernel here")


# speedup vs baseline: 2.3519x; 2.3519x over previous
import jax
import jax.numpy as jnp
from jax.experimental import pallas as pl
from jax.experimental.pallas import tpu as pltpu

_TILE = 256


def _vq_kernel(x_ref, ct_ref, o_ref, c2_ref):
    @pl.when(pl.program_id(0) == 0)
    def _():
        ct = ct_ref[...]
        c2_ref[...] = jnp.sum(ct * ct, axis=0, keepdims=True)

    dots = jnp.dot(x_ref[...], ct_ref[...], preferred_element_type=jnp.float32)
    s = c2_ref[...] - 2.0 * dots
    idx = jnp.argmin(s, axis=1).astype(jnp.int32)
    o_ref[...] = idx.reshape(1, 1, _TILE)


def kernel(inp, clusters):
    B, T, D = inp.shape
    C = clusters.shape[0]
    x = inp.reshape(B * T, D)
    ct = clusters.T
    nt = (B * T) // _TILE
    out = pl.pallas_call(
        _vq_kernel,
        grid=(nt,),
        in_specs=[
            pl.BlockSpec((_TILE, D), lambda i: (i, 0)),
            pl.BlockSpec((D, C), lambda i: (0, 0)),
        ],
        out_specs=pl.BlockSpec((1, 1, _TILE), lambda i: (i, 0, 0)),
        out_shape=jax.ShapeDtypeStruct((nt, 1, _TILE), jnp.int32),
        scratch_shapes=[pltpu.VMEM((1, C), jnp.float32)],
    )(x, ct)
    return out.reshape(B, T)
